# baseline (speedup 1.0000x reference)
import jax
import jax.numpy as jnp
from jax import lax
from jax.experimental import pallas as pl
from jax.experimental.pallas import tpu as pltpu
from jax.experimental.pallas import tpu_sc as plsc

_NUM_CLASSES = 1000000
_FEAT_DIM = 64
_BATCH = 16384
_LAMBDA_C = 0.01
_EPS = 1e-12

_PAIR_W = 2 * _FEAT_DIM
_NC = 2
_NS = 16
_NW = _NC * _NS
_B_PER_W = _BATCH // _NW
_IDX_W = 128
_N_CHUNKS = _B_PER_W // _IDX_W

_TC_ROWS = 1024
_TC_STEPS = _BATCH // _TC_ROWS


def _sc_gather_body(idx_hbm, table_hbm, out_hbm, iv, rows_v, sem):
    wid = lax.axis_index("s") * _NC + lax.axis_index("c")
    base = wid * _B_PER_W

    pltpu.sync_copy(idx_hbm.at[pl.ds(wid * _N_CHUNKS, _N_CHUNKS), :], iv)

    for j in range(_N_CHUNKS):
        pltpu.async_copy(table_hbm.at[iv.at[j]],
                         rows_v.at[pl.ds(j * _IDX_W, _IDX_W), :], sem)
    for j in range(_N_CHUNKS):
        pltpu.make_async_copy(table_hbm.at[iv.at[j]],
                              rows_v.at[pl.ds(j * _IDX_W, _IDX_W), :],
                              sem).wait()

    pltpu.sync_copy(rows_v, out_hbm.at[pl.ds(base, _B_PER_W), :])


_sc_gather = pl.kernel(
    _sc_gather_body,
    out_type=jax.ShapeDtypeStruct((_BATCH, _PAIR_W), jnp.float32),
    mesh=plsc.VectorSubcoreMesh(core_axis_name="c", subcore_axis_name="s"),
    scratch_types=[
        pltpu.VMEM((_N_CHUNKS, _IDX_W), jnp.int32),
        pltpu.VMEM((_B_PER_W, _PAIR_W), jnp.float32),
        pltpu.SemaphoreType.DMA,
    ],
)


def _tc_loss_body(f_ref, g_ref, p_ref, o_ref):
    i = pl.program_id(0)
    f = f_ref[...]
    g = g_ref[...]
    p = p_ref[...]
    lo = g[:, :_FEAT_DIM]
    hi = g[:, _FEAT_DIM:]
    c = lo + (hi - lo) * p

    e2 = jnp.float32(_EPS * _EPS)
    ff = jnp.maximum(jnp.sum(f * f, axis=1, keepdims=True), e2)
    cc = jnp.maximum(jnp.sum(c * c, axis=1, keepdims=True), e2)
    fc = jnp.sum(f * c, axis=1, keepdims=True)
    dist = (jnp.sum(f * f, axis=1, keepdims=True) / ff
            + jnp.sum(c * c, axis=1, keepdims=True) / cc
            - 2.0 * fc * lax.rsqrt(ff * cc))
    partial = (jnp.sum(dist, axis=0, keepdims=True)
               * jnp.float32(_LAMBDA_C / _BATCH))

    @pl.when(i == 0)
    def _():
        o_ref[...] = jnp.zeros_like(o_ref)

    o_ref[...] += partial


_tc_loss = pl.pallas_call(
    _tc_loss_body,
    grid=(_TC_STEPS,),
    in_specs=[
        pl.BlockSpec((_TC_ROWS, _FEAT_DIM), lambda i: (i, 0)),
        pl.BlockSpec((_TC_ROWS, _PAIR_W), lambda i: (i, 0)),
        pl.BlockSpec((_TC_ROWS, 1), lambda i: (i, 0)),
    ],
    out_specs=pl.BlockSpec((1, 1), lambda i: (0, 0)),
    out_shape=jax.ShapeDtypeStruct((1, 1), jnp.float32),
)


def kernel(features, labels, centers):
    labels32 = labels.astype(jnp.int32)
    pair_idx = (labels32 >> 1).reshape(_NW * _N_CHUNKS, _IDX_W)
    parity = (labels32 & 1).astype(jnp.float32).reshape(_BATCH, 1)
    table = centers.reshape(_NUM_CLASSES // 2, _PAIR_W)
    gathered = _sc_gather(pair_idx, table)
    return _tc_loss(features, gathered, parity)[0, 0]
